# baseline (speedup 1.0000x reference)
import functools
import jax
import jax.numpy as jnp
from jax import lax
from jax.experimental import pallas as pl
from jax.experimental.pallas import tpu as pltpu
from jax.experimental.pallas import tpu_sc as plsc

_K = 51
_PENALTY = 10.0
_LEVELS = ((22, 8), (14, 8), (6, 8), (0, 6))
_NW = 32


_UNROLL = 8
_NSUB = 4
_TAB = 256 * 16


def _sc_select_body(ma, dist_hbm, outt_hbm, outj_hbm, buf, hist, cbits, ccol,
                    tv, jv):
    _, N = dist_hbm.shape
    rows_per_w = ma // _NW
    nblk = rows_per_w // 16
    wid = lax.axis_index("s") * 2 + lax.axis_index("c")
    base = wid * rows_per_w
    lanes = lax.iota(jnp.int32, 16)
    ones = jnp.ones((16,), jnp.int32)
    zeros = jnp.zeros((16,), jnp.int32)
    rowbase = lanes * N

    def hist_scan(nb, kk, ntab):
        def scan(c, carry):
            total, found, bb, cntlt = carry
            hv = hist[pl.ds(c * 16, 16)]
            for tbl in range(1, ntab):
                hv = hv + hist[pl.ds(tbl * _TAB + c * 16, 16)]
            ntot = total + hv
            crossed = jnp.logical_and(jnp.logical_not(found), ntot >= kk)
            bb = jnp.where(crossed, zeros + c, bb)
            cntlt = jnp.where(crossed, total, cntlt)
            found = jnp.logical_or(found, crossed)
            return ntot, found, bb, cntlt

        _, _, bb, cntlt = lax.fori_loop(0, nb, scan,
                                        (zeros, zeros > 0, zeros, zeros))
        return bb, cntlt

    def clear_hist(nwords):
        def clr(c, _):
            for u in range(_UNROLL):
                hist[pl.ds((c * _UNROLL + u) * 16, 16)] = zeros
            return 0

        lax.fori_loop(0, nwords // (16 * _UNROLL), clr, 0)

    def do_block(b, _):
        row0 = base + b * 16
        pltpu.sync_copy(dist_hbm.at[pl.ds(row0, 16)], buf)

        def colbits(j):
            v = plsc.load_gather(buf, [lanes, zeros + j])
            return lax.bitcast_convert_type(v, jnp.int32)

        s0, w0l = _LEVELS[0]
        nb0 = 1 << w0l
        clear_hist(nb0 * 16 * _NSUB)

        def hpass(j8, _):
            for u in range(_UNROLL):
                j = j8 * _UNROLL + u
                bucket = lax.shift_right_logical(colbits(j), s0)
                plsc.addupdate_scatter(
                    hist, [(u % _NSUB) * _TAB + bucket * 16 + lanes], ones)
            return 0

        lax.fori_loop(0, N // _UNROLL, hpass, 0)
        bb, cntlt = hist_scan(nb0, zeros + _K, _NSUB)
        prefix = bb << s0
        kk = (zeros + _K) - cntlt

        def cpass(j4, ptr):
            for u in range(_UNROLL):
                j = j4 * _UNROLL + u
                bits = colbits(j)
                m = lax.shift_right_logical(bits, s0) == bb
                plsc.store_scatter(cbits, [rowbase + ptr], bits, mask=m)
                plsc.store_scatter(ccol, [rowbase + ptr], zeros + j, mask=m)
                ptr = ptr + m.astype(jnp.int32)
            return ptr

        cnt = lax.fori_loop(0, N // _UNROLL, cpass, zeros)
        maxcnt = jnp.max(cnt)

        def candbits(i):
            return plsc.load_gather(cbits, [rowbase + i])

        for (s, w) in _LEVELS[1:]:
            nb = 1 << w
            himask = jnp.int32(~((1 << (s + w)) - 1))
            pfx_hi = prefix & himask
            clear_hist(nb * 16)

            def hpass2(i, _):
                bits = candbits(i)
                match = jnp.logical_and((bits & himask) == pfx_hi, i < cnt)
                bucket = (bits >> s) & (nb - 1)
                plsc.addupdate_scatter(hist, [bucket * 16 + lanes], ones,
                                       mask=match)
                return 0

            lax.fori_loop(0, maxcnt, hpass2, 0)
            bb2, cntlt = hist_scan(nb, kk, 1)
            prefix = prefix | (bb2 << s)
            kk = kk - cntlt

        tbits = prefix
        extra = kk

        def jpass(i, carry):
            ec, jj = carry
            m = jnp.logical_and(candbits(i) == tbits, i < cnt)
            nec = ec + m.astype(jnp.int32)
            hit = jnp.logical_and(m, nec == extra)
            cc = plsc.load_gather(ccol, [rowbase + i])
            jj = jnp.where(hit, cc, jj)
            return nec, jj

        _, jj = lax.fori_loop(0, maxcnt, jpass, (zeros, zeros))
        tv[pl.ds(b * 16, 16)] = tbits
        jv[pl.ds(b * 16, 16)] = jj
        return 0

    lax.fori_loop(0, nblk, do_block, 0)
    pltpu.sync_copy(tv, outt_hbm.at[pl.ds(base, rows_per_w)])
    pltpu.sync_copy(jv, outj_hbm.at[pl.ds(base, rows_per_w)])


def _mlp_scalars(wl_ref, bl_ref, wg_ref, bg_ref):
    wg = wg_ref[...]
    wl = wl_ref[...]
    bl = bl_ref[...]
    emb = wl.shape[1]
    wg_mid = wg[:, 2:2 + emb]
    c0 = jnp.sum(wl[0:1, :] * wg_mid) / _K
    c1 = jnp.sum(wl[1:2, :] * wg_mid) / _K
    cb = jnp.sum(bl * wg_mid)
    return (c0, c1, cb, wg[0, 0], wg[0, 1], wg[0, 2 + emb], wg[0, 3 + emb],
            bg_ref[0, 0])


def _masked_output(d, t, ins, m_sel, scal, out_ref):
    c0, c1, cb, w0, w1, wi0, wi1, bg = scal
    mf = m_sel.astype(jnp.float32)
    sum_t = jnp.sum(t * mf, axis=1, keepdims=True)
    sum_d = jnp.sum(d * mf, axis=1, keepdims=True)
    c_row = (sum_t * c0 + sum_d * c1 + cb + bg
             + ins[:, 0:1] * wi0 + ins[:, 1:2] * wi1)
    out_ref[...] = jnp.where(m_sel, t * w0 + d * (w1 - 1.0) + c_row, _PENALTY)


def _tc_body(dist_ref, theta_ref, ins_ref, tb_ref, jj_ref, wl_ref, bl_ref,
             wg_ref, bg_ref, alias_ref, out_ref):
    del alias_ref
    d = dist_ref[...]
    t = theta_ref[...]
    R, N = d.shape
    scal = _mlp_scalars(wl_ref, bl_ref, wg_ref, bg_ref)
    thr = lax.bitcast_convert_type(tb_ref[...], jnp.float32)
    jv = jj_ref[...]
    col = lax.broadcasted_iota(jnp.int32, (R, N), 1)
    m_sel = (d < thr) | ((d == thr) & (col <= jv))
    _masked_output(d, t, ins_ref[...], m_sel, scal, out_ref)


def _tc_full_body(dist_ref, theta_ref, ins_ref, wl_ref, bl_ref,
                  wg_ref, bg_ref, out_ref):
    d = dist_ref[...]
    t = theta_ref[...]
    R, N = d.shape
    scal = _mlp_scalars(wl_ref, bl_ref, wg_ref, bg_ref)

    bits = lax.bitcast_convert_type(d, jnp.int32)
    lo = jnp.zeros((R, 1), jnp.int32)
    hi = jnp.full((R, 1), 0x3F800000, jnp.int32)

    def bstep(_, carry):
        lo, hi = carry
        mid = lax.shift_right_logical(lo + hi, 1)
        cnt = jnp.sum((bits <= mid).astype(jnp.int32), axis=1, keepdims=True)
        ge = cnt >= _K
        return jnp.where(ge, lo, mid + 1), jnp.where(ge, mid, hi)

    lo, hi = lax.fori_loop(0, 30, bstep, (lo, hi))
    tbits = hi
    thr = lax.bitcast_convert_type(tbits, jnp.float32)

    m_lt = d < thr
    m_eq = bits == tbits
    cnt_lt = jnp.sum(m_lt.astype(jnp.int32), axis=1, keepdims=True)
    extra = _K - cnt_lt
    col = lax.broadcasted_iota(jnp.int32, (R, N), 1)
    jlo = jnp.zeros((R, 1), jnp.int32)
    jhi = jnp.full((R, 1), N - 1, jnp.int32)

    def jstep(_, carry):
        jlo, jhi = carry
        mid = lax.shift_right_logical(jlo + jhi, 1)
        cnt = jnp.sum((m_eq & (col <= mid)).astype(jnp.int32), axis=1,
                      keepdims=True)
        ge = cnt >= extra
        return jnp.where(ge, jlo, mid + 1), jnp.where(ge, mid, jhi)

    jlo, jhi = lax.fori_loop(0, max(1, (N - 1).bit_length()), jstep,
                             (jlo, jhi))
    m_sel = m_lt | (m_eq & (col <= jhi))
    _masked_output(d, t, ins_ref[...], m_sel, scal, out_ref)


def kernel(theta, dist, ins_feature, W_local, b_local, W_global, b_global):
    B, N, _ = dist.shape
    M = B * N
    d2 = dist.reshape(M, N)
    t2 = theta.reshape(M, N)

    MA = (M * 4 // 16) // 512 * 512
    MA = max(512, MA) if M >= 512 * 2 else 0
    rows_per_w = MA // _NW

    ins2 = jnp.concatenate([ins_feature[0], ins_feature[1]],
                           axis=-1).reshape(M, 2)
    emb = W_local.shape[1]
    bl = b_local.reshape(1, emb)
    wg = W_global.reshape(1, emb + 4)
    bg = b_global.reshape(1, 1)
    R = 256
    while M % R != 0 or (MA and MA % R != 0):
        R //= 2

    if MA == 0:
        out2 = pl.pallas_call(
            _tc_full_body,
            grid=(M // R,),
            in_specs=[
                pl.BlockSpec((R, N), lambda i: (i, 0)),
                pl.BlockSpec((R, N), lambda i: (i, 0)),
                pl.BlockSpec((R, 2), lambda i: (i, 0)),
                pl.BlockSpec((2, emb), lambda i: (0, 0)),
                pl.BlockSpec((1, emb), lambda i: (0, 0)),
                pl.BlockSpec((1, emb + 4), lambda i: (0, 0)),
                pl.BlockSpec((1, 1), lambda i: (0, 0)),
            ],
            out_specs=pl.BlockSpec((R, N), lambda i: (i, 0)),
            out_shape=jax.ShapeDtypeStruct((M, N), jnp.float32),
        )(d2, t2, ins2, W_local, bl, wg, bg)
        return out2.reshape(B, N, N)

    sc_select = pl.kernel(
        functools.partial(_sc_select_body, MA),
        out_type=[jax.ShapeDtypeStruct((MA,), jnp.int32),
                  jax.ShapeDtypeStruct((MA,), jnp.int32)],
        mesh=plsc.VectorSubcoreMesh(core_axis_name="c", subcore_axis_name="s",
                                    num_cores=2, num_subcores=16),
        compiler_params=pltpu.CompilerParams(use_tc_tiling_on_sc=False,
                                             needs_layout_passes=False),
        scratch_types=[
            pltpu.VMEM((16, N), jnp.float32),
            pltpu.VMEM((256 * 16 * _NSUB,), jnp.int32),
            pltpu.VMEM((16 * N,), jnp.int32),
            pltpu.VMEM((16 * N,), jnp.int32),
            pltpu.VMEM((rows_per_w,), jnp.int32),
            pltpu.VMEM((rows_per_w,), jnp.int32),
        ],
    )
    tb, jj = sc_select(d2)

    off = MA // R
    out_partial = pl.pallas_call(
        _tc_full_body,
        grid=(M // R - off,),
        in_specs=[
            pl.BlockSpec((R, N), lambda i: (i + off, 0)),
            pl.BlockSpec((R, N), lambda i: (i + off, 0)),
            pl.BlockSpec((R, 2), lambda i: (i + off, 0)),
            pl.BlockSpec((2, emb), lambda i: (0, 0)),
            pl.BlockSpec((1, emb), lambda i: (0, 0)),
            pl.BlockSpec((1, emb + 4), lambda i: (0, 0)),
            pl.BlockSpec((1, 1), lambda i: (0, 0)),
        ],
        out_specs=pl.BlockSpec((R, N), lambda i: (i + off, 0)),
        out_shape=jax.ShapeDtypeStruct((M, N), jnp.float32),
    )(d2, t2, ins2, W_local, bl, wg, bg)

    out2 = pl.pallas_call(
        _tc_body,
        grid=(off,),
        in_specs=[
            pl.BlockSpec((R, N), lambda i: (i, 0)),
            pl.BlockSpec((R, N), lambda i: (i, 0)),
            pl.BlockSpec((R, 2), lambda i: (i, 0)),
            pl.BlockSpec((R, 1), lambda i: (i, 0)),
            pl.BlockSpec((R, 1), lambda i: (i, 0)),
            pl.BlockSpec((2, emb), lambda i: (0, 0)),
            pl.BlockSpec((1, emb), lambda i: (0, 0)),
            pl.BlockSpec((1, emb + 4), lambda i: (0, 0)),
            pl.BlockSpec((1, 1), lambda i: (0, 0)),
            pl.BlockSpec((8, 128), lambda i: (0, 0)),
        ],
        out_specs=pl.BlockSpec((R, N), lambda i: (i, 0)),
        out_shape=jax.ShapeDtypeStruct((M, N), jnp.float32),
        input_output_aliases={9: 0},
    )(d2, t2, ins2, tb.reshape(MA, 1), jj.reshape(MA, 1), W_local, bl, wg,
      bg, out_partial)
    return out2.reshape(B, N, N)


# speedup vs baseline: 1.0496x; 1.0496x over previous
import functools
import jax
import jax.numpy as jnp
from jax import lax
from jax.experimental import pallas as pl
from jax.experimental.pallas import tpu as pltpu
from jax.experimental.pallas import tpu_sc as plsc

_K = 51
_PENALTY = 10.0
_LEVELS = ((22, 8), (14, 8), (6, 8), (0, 6))
_NW = 32


_UNROLL = 8
_NSUB = 4
_TAB = 256 * 16


def _sc_select_body(ma, dist_hbm, outt_hbm, outj_hbm, buf, hist, cbits, ccol,
                    tv, jv):
    _, N = dist_hbm.shape
    rows_per_w = ma // _NW
    nblk = rows_per_w // 16
    wid = lax.axis_index("s") * 2 + lax.axis_index("c")
    base = wid * rows_per_w
    lanes = lax.iota(jnp.int32, 16)
    ones = jnp.ones((16,), jnp.int32)
    zeros = jnp.zeros((16,), jnp.int32)
    rowbase = lanes * N

    def hist_scan(nb, kk, ntab):
        def scan(c, carry):
            total, found, bb, cntlt = carry
            hv = hist[pl.ds(c * 16, 16)]
            for tbl in range(1, ntab):
                hv = hv + hist[pl.ds(tbl * _TAB + c * 16, 16)]
            ntot = total + hv
            crossed = jnp.logical_and(jnp.logical_not(found), ntot >= kk)
            bb = jnp.where(crossed, zeros + c, bb)
            cntlt = jnp.where(crossed, total, cntlt)
            found = jnp.logical_or(found, crossed)
            return ntot, found, bb, cntlt

        _, _, bb, cntlt = lax.fori_loop(0, nb, scan,
                                        (zeros, zeros > 0, zeros, zeros))
        return bb, cntlt

    def clear_hist(nwords):
        def clr(c, _):
            for u in range(_UNROLL):
                hist[pl.ds((c * _UNROLL + u) * 16, 16)] = zeros
            return 0

        lax.fori_loop(0, nwords // (16 * _UNROLL), clr, 0)

    def do_block(b, _):
        row0 = base + b * 16
        pltpu.sync_copy(dist_hbm.at[pl.ds(row0, 16)], buf)

        def colbits(j):
            v = plsc.load_gather(buf, [lanes, zeros + j])
            return lax.bitcast_convert_type(v, jnp.int32)

        s0, w0l = _LEVELS[0]
        nb0 = 1 << w0l
        clear_hist(nb0 * 16 * _NSUB)

        def hpass(j8, _):
            for u in range(_UNROLL):
                j = j8 * _UNROLL + u
                bucket = lax.shift_right_logical(colbits(j), s0)
                plsc.addupdate_scatter(
                    hist, [(u % _NSUB) * _TAB + bucket * 16 + lanes], ones)
            return 0

        lax.fori_loop(0, N // _UNROLL, hpass, 0)
        bb, cntlt = hist_scan(nb0, zeros + _K, _NSUB)
        prefix = bb << s0
        kk = (zeros + _K) - cntlt

        def cpass(j4, ptr):
            for u in range(_UNROLL):
                j = j4 * _UNROLL + u
                bits = colbits(j)
                m = lax.shift_right_logical(bits, s0) == bb
                plsc.store_scatter(cbits, [rowbase + ptr], bits, mask=m)
                plsc.store_scatter(ccol, [rowbase + ptr], zeros + j, mask=m)
                ptr = ptr + m.astype(jnp.int32)
            return ptr

        cnt = lax.fori_loop(0, N // _UNROLL, cpass, zeros)
        maxcnt = jnp.max(cnt)

        def candbits(i):
            return plsc.load_gather(cbits, [rowbase + i])

        for (s, w) in _LEVELS[1:]:
            nb = 1 << w
            himask = jnp.int32(~((1 << (s + w)) - 1))
            pfx_hi = prefix & himask
            clear_hist(nb * 16)

            def hpass2(i, _):
                bits = candbits(i)
                match = jnp.logical_and((bits & himask) == pfx_hi, i < cnt)
                bucket = (bits >> s) & (nb - 1)
                plsc.addupdate_scatter(hist, [bucket * 16 + lanes], ones,
                                       mask=match)
                return 0

            lax.fori_loop(0, maxcnt, hpass2, 0)
            bb2, cntlt = hist_scan(nb, kk, 1)
            prefix = prefix | (bb2 << s)
            kk = kk - cntlt

        tbits = prefix
        extra = kk

        def jpass(i, carry):
            ec, jj = carry
            m = jnp.logical_and(candbits(i) == tbits, i < cnt)
            nec = ec + m.astype(jnp.int32)
            hit = jnp.logical_and(m, nec == extra)
            cc = plsc.load_gather(ccol, [rowbase + i])
            jj = jnp.where(hit, cc, jj)
            return nec, jj

        _, jj = lax.fori_loop(0, maxcnt, jpass, (zeros, zeros))
        tv[pl.ds(b * 16, 16)] = tbits
        jv[pl.ds(b * 16, 16)] = jj
        return 0

    lax.fori_loop(0, nblk, do_block, 0)
    pltpu.sync_copy(tv, outt_hbm.at[pl.ds(base, rows_per_w)])
    pltpu.sync_copy(jv, outj_hbm.at[pl.ds(base, rows_per_w)])


def _mlp_scalars(wl_ref, bl_ref, wg_ref, bg_ref):
    wg = wg_ref[...]
    wl = wl_ref[...]
    bl = bl_ref[...]
    emb = wl.shape[1]
    wg_mid = wg[:, 2:2 + emb]
    c0 = jnp.sum(wl[0:1, :] * wg_mid) / _K
    c1 = jnp.sum(wl[1:2, :] * wg_mid) / _K
    cb = jnp.sum(bl * wg_mid)
    return (c0, c1, cb, wg[0, 0], wg[0, 1], wg[0, 2 + emb], wg[0, 3 + emb],
            bg_ref[0, 0])


def _masked_output(d, t, ins, m_sel, scal, out_ref):
    c0, c1, cb, w0, w1, wi0, wi1, bg = scal
    mf = m_sel.astype(jnp.float32)
    sum_t = jnp.sum(t * mf, axis=1, keepdims=True)
    sum_d = jnp.sum(d * mf, axis=1, keepdims=True)
    c_row = (sum_t * c0 + sum_d * c1 + cb + bg
             + ins[:, 0:1] * wi0 + ins[:, 1:2] * wi1)
    out_ref[...] = jnp.where(m_sel, t * w0 + d * (w1 - 1.0) + c_row, _PENALTY)


def _tc_body(dist_ref, theta_ref, ins_ref, tb_ref, jj_ref, wl_ref, bl_ref,
             wg_ref, bg_ref, alias_ref, out_ref):
    del alias_ref
    d = dist_ref[...]
    t = theta_ref[...]
    R, N = d.shape
    scal = _mlp_scalars(wl_ref, bl_ref, wg_ref, bg_ref)
    thr = lax.bitcast_convert_type(tb_ref[...], jnp.float32)
    jv = jj_ref[...]
    col = lax.broadcasted_iota(jnp.int32, (R, N), 1)
    m_sel = (d < thr) | ((d == thr) & (col <= jv))
    _masked_output(d, t, ins_ref[...], m_sel, scal, out_ref)


def _tc_full_body(dist_ref, theta_ref, ins_ref, wl_ref, bl_ref,
                  wg_ref, bg_ref, out_ref):
    d = dist_ref[...]
    t = theta_ref[...]
    R, N = d.shape
    scal = _mlp_scalars(wl_ref, bl_ref, wg_ref, bg_ref)

    bits = lax.bitcast_convert_type(d, jnp.int32)
    lo = jnp.zeros((R, 1), jnp.int32)
    hi = jnp.full((R, 1), 0x3F800000, jnp.int32)

    def bstep(_, carry):
        lo, hi = carry
        mid = lax.shift_right_logical(lo + hi, 1)
        cnt = jnp.sum((bits <= mid).astype(jnp.int32), axis=1, keepdims=True)
        ge = cnt >= _K
        return jnp.where(ge, lo, mid + 1), jnp.where(ge, mid, hi)

    lo, hi = lax.fori_loop(0, 30, bstep, (lo, hi))
    tbits = hi
    thr = lax.bitcast_convert_type(tbits, jnp.float32)

    m_lt = d < thr
    m_eq = bits == tbits
    cnt_lt = jnp.sum(m_lt.astype(jnp.int32), axis=1, keepdims=True)
    extra = _K - cnt_lt
    col = lax.broadcasted_iota(jnp.int32, (R, N), 1)
    jlo = jnp.zeros((R, 1), jnp.int32)
    jhi = jnp.full((R, 1), N - 1, jnp.int32)

    def jstep(_, carry):
        jlo, jhi = carry
        mid = lax.shift_right_logical(jlo + jhi, 1)
        cnt = jnp.sum((m_eq & (col <= mid)).astype(jnp.int32), axis=1,
                      keepdims=True)
        ge = cnt >= extra
        return jnp.where(ge, jlo, mid + 1), jnp.where(ge, mid, jhi)

    jlo, jhi = lax.fori_loop(0, max(1, (N - 1).bit_length()), jstep,
                             (jlo, jhi))
    m_sel = m_lt | (m_eq & (col <= jhi))
    _masked_output(d, t, ins_ref[...], m_sel, scal, out_ref)


def kernel(theta, dist, ins_feature, W_local, b_local, W_global, b_global):
    B, N, _ = dist.shape
    M = B * N
    d2 = dist.reshape(M, N)
    t2 = theta.reshape(M, N)

    MA = (M * 4 // 16) // 512 * 512
    MA = max(512, MA) if M >= 512 * 2 else 0
    rows_per_w = MA // _NW

    ins2 = jnp.concatenate([ins_feature[0], ins_feature[1]],
                           axis=-1).reshape(M, 2)
    emb = W_local.shape[1]
    bl = b_local.reshape(1, emb)
    wg = W_global.reshape(1, emb + 4)
    bg = b_global.reshape(1, 1)
    R = 256
    while M % R != 0 or (MA and MA % R != 0):
        R //= 2

    if MA == 0:
        out2 = pl.pallas_call(
            _tc_full_body,
            grid=(M // R,),
            in_specs=[
                pl.BlockSpec((R, N), lambda i: (i, 0)),
                pl.BlockSpec((R, N), lambda i: (i, 0)),
                pl.BlockSpec((R, 2), lambda i: (i, 0)),
                pl.BlockSpec((2, emb), lambda i: (0, 0)),
                pl.BlockSpec((1, emb), lambda i: (0, 0)),
                pl.BlockSpec((1, emb + 4), lambda i: (0, 0)),
                pl.BlockSpec((1, 1), lambda i: (0, 0)),
            ],
            out_specs=pl.BlockSpec((R, N), lambda i: (i, 0)),
            out_shape=jax.ShapeDtypeStruct((M, N), jnp.float32),
        )(d2, t2, ins2, W_local, bl, wg, bg)
        return out2.reshape(B, N, N)

    sc_select = pl.kernel(
        functools.partial(_sc_select_body, MA),
        out_type=[jax.ShapeDtypeStruct((MA,), jnp.int32),
                  jax.ShapeDtypeStruct((MA,), jnp.int32)],
        mesh=plsc.VectorSubcoreMesh(core_axis_name="c", subcore_axis_name="s",
                                    num_cores=2, num_subcores=16),
        compiler_params=pltpu.CompilerParams(use_tc_tiling_on_sc=False,
                                             needs_layout_passes=False),
        scratch_types=[
            pltpu.VMEM((16, N), jnp.float32),
            pltpu.VMEM((256 * 16 * _NSUB,), jnp.int32),
            pltpu.VMEM((16 * N,), jnp.int32),
            pltpu.VMEM((16 * N,), jnp.int32),
            pltpu.VMEM((rows_per_w,), jnp.int32),
            pltpu.VMEM((rows_per_w,), jnp.int32),
        ],
    )
    tb, jj = sc_select(d2[:MA])

    off = MA // R
    out_partial = pl.pallas_call(
        _tc_full_body,
        grid=(M // R - off,),
        in_specs=[
            pl.BlockSpec((R, N), lambda i: (i + off, 0)),
            pl.BlockSpec((R, N), lambda i: (i + off, 0)),
            pl.BlockSpec((R, 2), lambda i: (i + off, 0)),
            pl.BlockSpec((2, emb), lambda i: (0, 0)),
            pl.BlockSpec((1, emb), lambda i: (0, 0)),
            pl.BlockSpec((1, emb + 4), lambda i: (0, 0)),
            pl.BlockSpec((1, 1), lambda i: (0, 0)),
        ],
        out_specs=pl.BlockSpec((R, N), lambda i: (i + off, 0)),
        out_shape=jax.ShapeDtypeStruct((M, N), jnp.float32),
    )(d2, t2, ins2, W_local, bl, wg, bg)

    out2 = pl.pallas_call(
        _tc_body,
        grid=(off,),
        in_specs=[
            pl.BlockSpec((R, N), lambda i: (i, 0)),
            pl.BlockSpec((R, N), lambda i: (i, 0)),
            pl.BlockSpec((R, 2), lambda i: (i, 0)),
            pl.BlockSpec((R, 1), lambda i: (i, 0)),
            pl.BlockSpec((R, 1), lambda i: (i, 0)),
            pl.BlockSpec((2, emb), lambda i: (0, 0)),
            pl.BlockSpec((1, emb), lambda i: (0, 0)),
            pl.BlockSpec((1, emb + 4), lambda i: (0, 0)),
            pl.BlockSpec((1, 1), lambda i: (0, 0)),
            pl.BlockSpec((8, 128), lambda i: (0, 0)),
        ],
        out_specs=pl.BlockSpec((R, N), lambda i: (i, 0)),
        out_shape=jax.ShapeDtypeStruct((M, N), jnp.float32),
        input_output_aliases={9: 0},
    )(d2, t2, ins2, tb.reshape(MA, 1), jj.reshape(MA, 1), W_local, bl, wg,
      bg, out_partial)
    return out2.reshape(B, N, N)


# speedup vs baseline: 1.1015x; 1.0495x over previous
import functools
import jax
import jax.numpy as jnp
from jax import lax
from jax.experimental import pallas as pl
from jax.experimental.pallas import tpu as pltpu
from jax.experimental.pallas import tpu_sc as plsc

_K = 51
_PENALTY = 10.0
_LEVELS = ((22, 8), (14, 8), (6, 8), (0, 6))
_NW = 32


_UNROLL = 8
_NSUB = 4
_TAB = 256 * 16


def _sc_select_body(ma, dist_hbm, outt_hbm, outj_hbm, buf, hist, cbits, ccol,
                    tv, jv):
    _, N = dist_hbm.shape
    rows_per_w = ma // _NW
    nblk = rows_per_w // 16
    wid = lax.axis_index("s") * 2 + lax.axis_index("c")
    base = wid * rows_per_w
    lanes = lax.iota(jnp.int32, 16)
    ones = jnp.ones((16,), jnp.int32)
    zeros = jnp.zeros((16,), jnp.int32)
    rowbase = lanes * N

    def hist_scan(nb, kk, ntab):
        @plsc.parallel_loop(0, nb, carry=(zeros, zeros > 0, zeros, zeros),
                            unroll=4)
        def scan(c, carry):
            total, found, bb, cntlt = carry
            hv = hist[pl.ds(c * 16, 16)]
            for tbl in range(1, ntab):
                hv = hv + hist[pl.ds(tbl * _TAB + c * 16, 16)]
            ntot = total + hv
            crossed = jnp.logical_and(jnp.logical_not(found), ntot >= kk)
            bb = jnp.where(crossed, zeros + c, bb)
            cntlt = jnp.where(crossed, total, cntlt)
            found = jnp.logical_or(found, crossed)
            return ntot, found, bb, cntlt

        _, _, bb, cntlt = scan
        return bb, cntlt

    def clear_hist(nwords):
        @plsc.parallel_loop(0, nwords // 16, step=_UNROLL)
        def clr(c):
            for u in range(_UNROLL):
                hist[pl.ds((c + u) * 16, 16)] = zeros

    def do_block(b, _):
        row0 = base + b * 16
        pltpu.sync_copy(dist_hbm.at[pl.ds(row0, 16)], buf)

        def colbits(j):
            v = plsc.load_gather(buf, [lanes, zeros + j])
            return lax.bitcast_convert_type(v, jnp.int32)

        s0, w0l = _LEVELS[0]
        nb0 = 1 << w0l
        clear_hist(nb0 * 16 * _NSUB)

        @plsc.parallel_loop(0, N, step=_UNROLL)
        def hpass(j8):
            for u in range(_UNROLL):
                j = j8 + u
                bucket = lax.shift_right_logical(colbits(j), s0)
                plsc.addupdate_scatter(
                    hist, [(u % _NSUB) * _TAB + bucket * 16 + lanes], ones)

        bb, cntlt = hist_scan(nb0, zeros + _K, _NSUB)
        prefix = bb << s0
        kk = (zeros + _K) - cntlt

        @plsc.parallel_loop(0, N, step=_UNROLL, carry=zeros)
        def cpass(j4, ptr):
            for u in range(_UNROLL):
                j = j4 + u
                bits = colbits(j)
                m = lax.shift_right_logical(bits, s0) == bb
                plsc.store_scatter(cbits, [rowbase + ptr], bits, mask=m)
                plsc.store_scatter(ccol, [rowbase + ptr], zeros + j, mask=m)
                ptr = ptr + m.astype(jnp.int32)
            return ptr

        cnt = cpass
        maxcnt = jnp.max(cnt)

        def candbits(i):
            return plsc.load_gather(cbits, [rowbase + i])

        for (s, w) in _LEVELS[1:]:
            nb = 1 << w
            himask = jnp.int32(~((1 << (s + w)) - 1))
            pfx_hi = prefix & himask
            clear_hist(nb * 16)

            def hpass2(i, _):
                bits = candbits(i)
                match = jnp.logical_and((bits & himask) == pfx_hi, i < cnt)
                bucket = (bits >> s) & (nb - 1)
                plsc.addupdate_scatter(hist, [bucket * 16 + lanes], ones,
                                       mask=match)
                return 0

            lax.fori_loop(0, maxcnt, hpass2, 0)
            bb2, cntlt = hist_scan(nb, kk, 1)
            prefix = prefix | (bb2 << s)
            kk = kk - cntlt

        tbits = prefix
        extra = kk

        @plsc.parallel_loop(0, maxcnt, carry=(zeros, zeros), unroll=2)
        def jpass(i, carry):
            ec, jj = carry
            m = jnp.logical_and(candbits(i) == tbits, i < cnt)
            nec = ec + m.astype(jnp.int32)
            hit = jnp.logical_and(m, nec == extra)
            cc = plsc.load_gather(ccol, [rowbase + i])
            jj = jnp.where(hit, cc, jj)
            return nec, jj

        _, jj = jpass
        tv[pl.ds(b * 16, 16)] = tbits
        jv[pl.ds(b * 16, 16)] = jj
        return 0

    lax.fori_loop(0, nblk, do_block, 0)
    pltpu.sync_copy(tv, outt_hbm.at[pl.ds(base, rows_per_w)])
    pltpu.sync_copy(jv, outj_hbm.at[pl.ds(base, rows_per_w)])


def _mlp_scalars(wl_ref, bl_ref, wg_ref, bg_ref):
    wg = wg_ref[...]
    wl = wl_ref[...]
    bl = bl_ref[...]
    emb = wl.shape[1]
    wg_mid = wg[:, 2:2 + emb]
    c0 = jnp.sum(wl[0:1, :] * wg_mid) / _K
    c1 = jnp.sum(wl[1:2, :] * wg_mid) / _K
    cb = jnp.sum(bl * wg_mid)
    return (c0, c1, cb, wg[0, 0], wg[0, 1], wg[0, 2 + emb], wg[0, 3 + emb],
            bg_ref[0, 0])


def _masked_output(d, t, ins, m_sel, scal, out_ref):
    c0, c1, cb, w0, w1, wi0, wi1, bg = scal
    mf = m_sel.astype(jnp.float32)
    sum_t = jnp.sum(t * mf, axis=1, keepdims=True)
    sum_d = jnp.sum(d * mf, axis=1, keepdims=True)
    c_row = (sum_t * c0 + sum_d * c1 + cb + bg
             + ins[:, 0:1] * wi0 + ins[:, 1:2] * wi1)
    out_ref[...] = jnp.where(m_sel, t * w0 + d * (w1 - 1.0) + c_row, _PENALTY)


def _tc_body(dist_ref, theta_ref, ins_ref, tb_ref, jj_ref, wl_ref, bl_ref,
             wg_ref, bg_ref, alias_ref, out_ref):
    del alias_ref
    d = dist_ref[...]
    t = theta_ref[...]
    R, N = d.shape
    scal = _mlp_scalars(wl_ref, bl_ref, wg_ref, bg_ref)
    thr = lax.bitcast_convert_type(tb_ref[...], jnp.float32)
    jv = jj_ref[...]
    col = lax.broadcasted_iota(jnp.int32, (R, N), 1)
    m_sel = (d < thr) | ((d == thr) & (col <= jv))
    _masked_output(d, t, ins_ref[...], m_sel, scal, out_ref)


def _tc_full_body(dist_ref, theta_ref, ins_ref, wl_ref, bl_ref,
                  wg_ref, bg_ref, out_ref):
    d = dist_ref[...]
    t = theta_ref[...]
    R, N = d.shape
    scal = _mlp_scalars(wl_ref, bl_ref, wg_ref, bg_ref)

    bits = lax.bitcast_convert_type(d, jnp.int32)
    lo = jnp.zeros((R, 1), jnp.int32)
    hi = jnp.full((R, 1), 0x3F800000, jnp.int32)

    def bstep(_, carry):
        lo, hi = carry
        mid = lax.shift_right_logical(lo + hi, 1)
        cnt = jnp.sum((bits <= mid).astype(jnp.int32), axis=1, keepdims=True)
        ge = cnt >= _K
        return jnp.where(ge, lo, mid + 1), jnp.where(ge, mid, hi)

    lo, hi = lax.fori_loop(0, 30, bstep, (lo, hi))
    tbits = hi
    thr = lax.bitcast_convert_type(tbits, jnp.float32)

    m_lt = d < thr
    m_eq = bits == tbits
    cnt_lt = jnp.sum(m_lt.astype(jnp.int32), axis=1, keepdims=True)
    extra = _K - cnt_lt
    col = lax.broadcasted_iota(jnp.int32, (R, N), 1)
    jlo = jnp.zeros((R, 1), jnp.int32)
    jhi = jnp.full((R, 1), N - 1, jnp.int32)

    def jstep(_, carry):
        jlo, jhi = carry
        mid = lax.shift_right_logical(jlo + jhi, 1)
        cnt = jnp.sum((m_eq & (col <= mid)).astype(jnp.int32), axis=1,
                      keepdims=True)
        ge = cnt >= extra
        return jnp.where(ge, jlo, mid + 1), jnp.where(ge, mid, jhi)

    jlo, jhi = lax.fori_loop(0, max(1, (N - 1).bit_length()), jstep,
                             (jlo, jhi))
    m_sel = m_lt | (m_eq & (col <= jhi))
    _masked_output(d, t, ins_ref[...], m_sel, scal, out_ref)


def kernel(theta, dist, ins_feature, W_local, b_local, W_global, b_global):
    B, N, _ = dist.shape
    M = B * N
    d2 = dist.reshape(M, N)
    t2 = theta.reshape(M, N)

    MA = (M * 5 // 16) // 512 * 512
    MA = max(512, MA) if M >= 512 * 2 else 0
    rows_per_w = MA // _NW

    ins2 = jnp.concatenate([ins_feature[0], ins_feature[1]],
                           axis=-1).reshape(M, 2)
    emb = W_local.shape[1]
    bl = b_local.reshape(1, emb)
    wg = W_global.reshape(1, emb + 4)
    bg = b_global.reshape(1, 1)
    R = 256
    while M % R != 0 or (MA and MA % R != 0):
        R //= 2

    if MA == 0:
        out2 = pl.pallas_call(
            _tc_full_body,
            grid=(M // R,),
            in_specs=[
                pl.BlockSpec((R, N), lambda i: (i, 0)),
                pl.BlockSpec((R, N), lambda i: (i, 0)),
                pl.BlockSpec((R, 2), lambda i: (i, 0)),
                pl.BlockSpec((2, emb), lambda i: (0, 0)),
                pl.BlockSpec((1, emb), lambda i: (0, 0)),
                pl.BlockSpec((1, emb + 4), lambda i: (0, 0)),
                pl.BlockSpec((1, 1), lambda i: (0, 0)),
            ],
            out_specs=pl.BlockSpec((R, N), lambda i: (i, 0)),
            out_shape=jax.ShapeDtypeStruct((M, N), jnp.float32),
        )(d2, t2, ins2, W_local, bl, wg, bg)
        return out2.reshape(B, N, N)

    sc_select = pl.kernel(
        functools.partial(_sc_select_body, MA),
        out_type=[jax.ShapeDtypeStruct((MA,), jnp.int32),
                  jax.ShapeDtypeStruct((MA,), jnp.int32)],
        mesh=plsc.VectorSubcoreMesh(core_axis_name="c", subcore_axis_name="s",
                                    num_cores=2, num_subcores=16),
        compiler_params=pltpu.CompilerParams(use_tc_tiling_on_sc=False,
                                             needs_layout_passes=False),
        scratch_types=[
            pltpu.VMEM((16, N), jnp.float32),
            pltpu.VMEM((256 * 16 * _NSUB,), jnp.int32),
            pltpu.VMEM((16 * N,), jnp.int32),
            pltpu.VMEM((16 * N,), jnp.int32),
            pltpu.VMEM((rows_per_w,), jnp.int32),
            pltpu.VMEM((rows_per_w,), jnp.int32),
        ],
    )
    tb, jj = sc_select(d2[:MA])

    off = MA // R
    out_partial = pl.pallas_call(
        _tc_full_body,
        grid=(M // R - off,),
        in_specs=[
            pl.BlockSpec((R, N), lambda i: (i + off, 0)),
            pl.BlockSpec((R, N), lambda i: (i + off, 0)),
            pl.BlockSpec((R, 2), lambda i: (i + off, 0)),
            pl.BlockSpec((2, emb), lambda i: (0, 0)),
            pl.BlockSpec((1, emb), lambda i: (0, 0)),
            pl.BlockSpec((1, emb + 4), lambda i: (0, 0)),
            pl.BlockSpec((1, 1), lambda i: (0, 0)),
        ],
        out_specs=pl.BlockSpec((R, N), lambda i: (i + off, 0)),
        out_shape=jax.ShapeDtypeStruct((M, N), jnp.float32),
    )(d2, t2, ins2, W_local, bl, wg, bg)

    out2 = pl.pallas_call(
        _tc_body,
        grid=(off,),
        in_specs=[
            pl.BlockSpec((R, N), lambda i: (i, 0)),
            pl.BlockSpec((R, N), lambda i: (i, 0)),
            pl.BlockSpec((R, 2), lambda i: (i, 0)),
            pl.BlockSpec((R, 1), lambda i: (i, 0)),
            pl.BlockSpec((R, 1), lambda i: (i, 0)),
            pl.BlockSpec((2, emb), lambda i: (0, 0)),
            pl.BlockSpec((1, emb), lambda i: (0, 0)),
            pl.BlockSpec((1, emb + 4), lambda i: (0, 0)),
            pl.BlockSpec((1, 1), lambda i: (0, 0)),
            pl.BlockSpec((8, 128), lambda i: (0, 0)),
        ],
        out_specs=pl.BlockSpec((R, N), lambda i: (i, 0)),
        out_shape=jax.ShapeDtypeStruct((M, N), jnp.float32),
        input_output_aliases={9: 0},
    )(d2, t2, ins2, tb.reshape(MA, 1), jj.reshape(MA, 1), W_local, bl, wg,
      bg, out_partial)
    return out2.reshape(B, N, N)


# speedup vs baseline: 1.1627x; 1.0555x over previous
import functools
import jax
import jax.numpy as jnp
from jax import lax
from jax.experimental import pallas as pl
from jax.experimental.pallas import tpu as pltpu
from jax.experimental.pallas import tpu_sc as plsc

_K = 51
_PENALTY = 10.0
_LEVELS = ((22, 8), (14, 8), (6, 8), (0, 6))
_NW = 32


_UNROLL = 8
_NSUB = 4
_TAB = 256 * 16


def _sc_select_body(ma, dist_hbm, outt_hbm, outj_hbm, buf, hist, cbits, ccol,
                    tv, jv):
    _, N = dist_hbm.shape
    rows_per_w = ma // _NW
    nblk = rows_per_w // 16
    wid = lax.axis_index("s") * 2 + lax.axis_index("c")
    base = wid * rows_per_w
    lanes = lax.iota(jnp.int32, 16)
    ones = jnp.ones((16,), jnp.int32)
    zeros = jnp.zeros((16,), jnp.int32)
    rowbase = lanes * N

    def hist_scan(nb, kk, ntab):
        @plsc.parallel_loop(0, nb, carry=(zeros, zeros > 0, zeros, zeros),
                            unroll=4)
        def scan(c, carry):
            total, found, bb, cntlt = carry
            hv = hist[pl.ds(c * 16, 16)]
            for tbl in range(1, ntab):
                hv = hv + hist[pl.ds(tbl * _TAB + c * 16, 16)]
            ntot = total + hv
            crossed = jnp.logical_and(jnp.logical_not(found), ntot >= kk)
            bb = jnp.where(crossed, zeros + c, bb)
            cntlt = jnp.where(crossed, total, cntlt)
            found = jnp.logical_or(found, crossed)
            return ntot, found, bb, cntlt

        _, _, bb, cntlt = scan
        return bb, cntlt

    def clear_hist(nwords):
        @plsc.parallel_loop(0, nwords // 16, step=_UNROLL)
        def clr(c):
            for u in range(_UNROLL):
                hist[pl.ds((c + u) * 16, 16)] = zeros

    def do_block(b, _):
        row0 = base + b * 16
        pltpu.sync_copy(dist_hbm.at[pl.ds(row0, 16)], buf)

        def colbits(j):
            v = plsc.load_gather(buf, [lanes, zeros + j])
            return lax.bitcast_convert_type(v, jnp.int32)

        s0, w0l = _LEVELS[0]
        nb0 = 1 << w0l
        clear_hist(nb0 * 16 * _NSUB)

        @plsc.parallel_loop(0, N, step=_UNROLL)
        def hpass(j8):
            for u in range(_UNROLL):
                j = j8 + u
                bucket = lax.shift_right_logical(colbits(j), s0)
                plsc.addupdate_scatter(
                    hist, [(u % _NSUB) * _TAB + bucket * 16 + lanes], ones)

        bb, cntlt = hist_scan(nb0, zeros + _K, _NSUB)
        prefix = bb << s0
        kk = (zeros + _K) - cntlt

        @plsc.parallel_loop(0, N, step=_UNROLL, carry=zeros)
        def cpass(j4, ptr):
            for u in range(_UNROLL):
                j = j4 + u
                bits = colbits(j)
                m = lax.shift_right_logical(bits, s0) == bb
                plsc.store_scatter(cbits, [rowbase + ptr], bits, mask=m)
                plsc.store_scatter(ccol, [rowbase + ptr], zeros + j, mask=m)
                ptr = ptr + m.astype(jnp.int32)
            return ptr

        cnt = cpass
        maxcnt = jnp.max(cnt)

        def candbits(i):
            return plsc.load_gather(cbits, [rowbase + i])

        for (s, w) in _LEVELS[1:]:
            nb = 1 << w
            himask = jnp.int32(~((1 << (s + w)) - 1))
            pfx_hi = prefix & himask
            clear_hist(nb * 16)

            def hpass2(i, _):
                bits = candbits(i)
                match = jnp.logical_and((bits & himask) == pfx_hi, i < cnt)
                bucket = (bits >> s) & (nb - 1)
                plsc.addupdate_scatter(hist, [bucket * 16 + lanes], ones,
                                       mask=match)
                return 0

            lax.fori_loop(0, maxcnt, hpass2, 0)
            bb2, cntlt = hist_scan(nb, kk, 1)
            prefix = prefix | (bb2 << s)
            kk = kk - cntlt

        tbits = prefix
        extra = kk

        @plsc.parallel_loop(0, maxcnt, carry=(zeros, zeros), unroll=2)
        def jpass(i, carry):
            ec, jj = carry
            m = jnp.logical_and(candbits(i) == tbits, i < cnt)
            nec = ec + m.astype(jnp.int32)
            hit = jnp.logical_and(m, nec == extra)
            cc = plsc.load_gather(ccol, [rowbase + i])
            jj = jnp.where(hit, cc, jj)
            return nec, jj

        _, jj = jpass
        tv[pl.ds(b * 16, 16)] = tbits
        jv[pl.ds(b * 16, 16)] = jj
        return 0

    lax.fori_loop(0, nblk, do_block, 0)
    pltpu.sync_copy(tv, outt_hbm.at[pl.ds(base, rows_per_w)])
    pltpu.sync_copy(jv, outj_hbm.at[pl.ds(base, rows_per_w)])


def _mlp_scalars(wl_ref, bl_ref, wg_ref, bg_ref):
    wg = wg_ref[...]
    wl = wl_ref[...]
    bl = bl_ref[...]
    emb = wl.shape[1]
    wg_mid = wg[:, 2:2 + emb]
    c0 = jnp.sum(wl[0:1, :] * wg_mid) / _K
    c1 = jnp.sum(wl[1:2, :] * wg_mid) / _K
    cb = jnp.sum(bl * wg_mid)
    return (c0, c1, cb, wg[0, 0], wg[0, 1], wg[0, 2 + emb], wg[0, 3 + emb],
            bg_ref[0, 0])


def _masked_output(d, t, ins, m_sel, scal, out_ref):
    c0, c1, cb, w0, w1, wi0, wi1, bg = scal
    mf = m_sel.astype(jnp.float32)
    sum_t = jnp.sum(t * mf, axis=1, keepdims=True)
    sum_d = jnp.sum(d * mf, axis=1, keepdims=True)
    c_row = (sum_t * c0 + sum_d * c1 + cb + bg
             + ins[:, 0:1] * wi0 + ins[:, 1:2] * wi1)
    out_ref[...] = jnp.where(m_sel, t * w0 + d * (w1 - 1.0) + c_row, _PENALTY)


def _tc_body(dist_ref, theta_ref, ins_ref, tb_ref, jj_ref, wl_ref, bl_ref,
             wg_ref, bg_ref, alias_ref, out_ref):
    del alias_ref
    d = dist_ref[...]
    t = theta_ref[...]
    R, N = d.shape
    scal = _mlp_scalars(wl_ref, bl_ref, wg_ref, bg_ref)
    thr = lax.bitcast_convert_type(tb_ref[...], jnp.float32)
    jv = jj_ref[...]
    col = lax.broadcasted_iota(jnp.int32, (R, N), 1)
    m_sel = (d < thr) | ((d == thr) & (col <= jv))
    _masked_output(d, t, ins_ref[...], m_sel, scal, out_ref)


def _tc_full_body(dist_ref, theta_ref, ins_ref, wl_ref, bl_ref,
                  wg_ref, bg_ref, out_ref):
    d = dist_ref[...]
    t = theta_ref[...]
    R, N = d.shape
    scal = _mlp_scalars(wl_ref, bl_ref, wg_ref, bg_ref)

    bits = lax.bitcast_convert_type(d, jnp.int32)
    lo = jnp.zeros((R, 1), jnp.int32)
    hi = jnp.full((R, 1), 0x3F800000, jnp.int32)

    def bstep(_, carry):
        lo, hi = carry
        mid = lax.shift_right_logical(lo + hi, 1)
        cnt = jnp.sum((bits <= mid).astype(jnp.int32), axis=1, keepdims=True)
        ge = cnt >= _K
        return jnp.where(ge, lo, mid + 1), jnp.where(ge, mid, hi)

    lo, hi = lax.fori_loop(0, 30, bstep, (lo, hi))
    tbits = hi
    thr = lax.bitcast_convert_type(tbits, jnp.float32)

    m_lt = d < thr
    m_eq = bits == tbits
    cnt_lt = jnp.sum(m_lt.astype(jnp.int32), axis=1, keepdims=True)
    extra = _K - cnt_lt
    col = lax.broadcasted_iota(jnp.int32, (R, N), 1)
    jlo = jnp.zeros((R, 1), jnp.int32)
    jhi = jnp.full((R, 1), N - 1, jnp.int32)

    def jstep(_, carry):
        jlo, jhi = carry
        mid = lax.shift_right_logical(jlo + jhi, 1)
        cnt = jnp.sum((m_eq & (col <= mid)).astype(jnp.int32), axis=1,
                      keepdims=True)
        ge = cnt >= extra
        return jnp.where(ge, jlo, mid + 1), jnp.where(ge, mid, jhi)

    jlo, jhi = lax.fori_loop(0, max(1, (N - 1).bit_length()), jstep,
                             (jlo, jhi))
    m_sel = m_lt | (m_eq & (col <= jhi))
    _masked_output(d, t, ins_ref[...], m_sel, scal, out_ref)


def kernel(theta, dist, ins_feature, W_local, b_local, W_global, b_global):
    B, N, _ = dist.shape
    M = B * N
    d2 = dist.reshape(M, N)
    t2 = theta.reshape(M, N)

    MA = (M * 6 // 16) // 512 * 512
    MA = max(512, MA) if M >= 512 * 2 else 0
    rows_per_w = MA // _NW

    ins2 = jnp.concatenate([ins_feature[0], ins_feature[1]],
                           axis=-1).reshape(M, 2)
    emb = W_local.shape[1]
    bl = b_local.reshape(1, emb)
    wg = W_global.reshape(1, emb + 4)
    bg = b_global.reshape(1, 1)
    R = 256
    while M % R != 0 or (MA and MA % R != 0):
        R //= 2

    if MA == 0:
        out2 = pl.pallas_call(
            _tc_full_body,
            grid=(M // R,),
            in_specs=[
                pl.BlockSpec((R, N), lambda i: (i, 0)),
                pl.BlockSpec((R, N), lambda i: (i, 0)),
                pl.BlockSpec((R, 2), lambda i: (i, 0)),
                pl.BlockSpec((2, emb), lambda i: (0, 0)),
                pl.BlockSpec((1, emb), lambda i: (0, 0)),
                pl.BlockSpec((1, emb + 4), lambda i: (0, 0)),
                pl.BlockSpec((1, 1), lambda i: (0, 0)),
            ],
            out_specs=pl.BlockSpec((R, N), lambda i: (i, 0)),
            out_shape=jax.ShapeDtypeStruct((M, N), jnp.float32),
        )(d2, t2, ins2, W_local, bl, wg, bg)
        return out2.reshape(B, N, N)

    sc_select = pl.kernel(
        functools.partial(_sc_select_body, MA),
        out_type=[jax.ShapeDtypeStruct((MA,), jnp.int32),
                  jax.ShapeDtypeStruct((MA,), jnp.int32)],
        mesh=plsc.VectorSubcoreMesh(core_axis_name="c", subcore_axis_name="s",
                                    num_cores=2, num_subcores=16),
        compiler_params=pltpu.CompilerParams(use_tc_tiling_on_sc=False,
                                             needs_layout_passes=False),
        scratch_types=[
            pltpu.VMEM((16, N), jnp.float32),
            pltpu.VMEM((256 * 16 * _NSUB,), jnp.int32),
            pltpu.VMEM((16 * N,), jnp.int32),
            pltpu.VMEM((16 * N,), jnp.int32),
            pltpu.VMEM((rows_per_w,), jnp.int32),
            pltpu.VMEM((rows_per_w,), jnp.int32),
        ],
    )
    tb, jj = sc_select(d2[:MA])

    off = MA // R
    out_partial = pl.pallas_call(
        _tc_full_body,
        grid=(M // R - off,),
        in_specs=[
            pl.BlockSpec((R, N), lambda i: (i + off, 0)),
            pl.BlockSpec((R, N), lambda i: (i + off, 0)),
            pl.BlockSpec((R, 2), lambda i: (i + off, 0)),
            pl.BlockSpec((2, emb), lambda i: (0, 0)),
            pl.BlockSpec((1, emb), lambda i: (0, 0)),
            pl.BlockSpec((1, emb + 4), lambda i: (0, 0)),
            pl.BlockSpec((1, 1), lambda i: (0, 0)),
        ],
        out_specs=pl.BlockSpec((R, N), lambda i: (i + off, 0)),
        out_shape=jax.ShapeDtypeStruct((M, N), jnp.float32),
    )(d2, t2, ins2, W_local, bl, wg, bg)

    out2 = pl.pallas_call(
        _tc_body,
        grid=(off,),
        in_specs=[
            pl.BlockSpec((R, N), lambda i: (i, 0)),
            pl.BlockSpec((R, N), lambda i: (i, 0)),
            pl.BlockSpec((R, 2), lambda i: (i, 0)),
            pl.BlockSpec((R, 1), lambda i: (i, 0)),
            pl.BlockSpec((R, 1), lambda i: (i, 0)),
            pl.BlockSpec((2, emb), lambda i: (0, 0)),
            pl.BlockSpec((1, emb), lambda i: (0, 0)),
            pl.BlockSpec((1, emb + 4), lambda i: (0, 0)),
            pl.BlockSpec((1, 1), lambda i: (0, 0)),
            pl.BlockSpec((8, 128), lambda i: (0, 0)),
        ],
        out_specs=pl.BlockSpec((R, N), lambda i: (i, 0)),
        out_shape=jax.ShapeDtypeStruct((M, N), jnp.float32),
        input_output_aliases={9: 0},
    )(d2, t2, ins2, tb.reshape(MA, 1), jj.reshape(MA, 1), W_local, bl, wg,
      bg, out_partial)
    return out2.reshape(B, N, N)
